# baseline (speedup 1.0000x reference)
import numpy as np

import jax
import jax.numpy as jnp
from jax.experimental import pallas as pl
from jax.experimental.pallas import tpu as pltpu

F32 = jnp.float32
BF16 = jnp.bfloat16

H = 64
WP = H + 2
SP = H * WP
HALO = WP + 1
BUFW = SP + 2 * HALO
SCALES = (2, 4, 8, 16)
POFF = {2: 0, 4: 8, 8: 32, 16: 112}
PTOT = 400


def _avg_mat(in_size, out_size):
    m = np.zeros((out_size, in_size), np.float32)
    for i in range(out_size):
        start = (i * in_size) // out_size
        end = -(-((i + 1) * in_size) // out_size)
        m[i, start:end] = 1.0 / (end - start)
    return m


def _bil_mat(in_size, out_size):
    m = np.zeros((out_size, in_size), np.float32)
    if out_size == 1 or in_size == 1:
        m[:, 0] = 1.0
        return m
    scale = (in_size - 1) / (out_size - 1)
    for i in range(out_size):
        src = i * scale
        i0 = min(int(np.floor(src)), in_size - 1)
        i1 = min(i0 + 1, in_size - 1)
        w1 = src - i0
        m[i, i0] += 1.0 - w1
        m[i, i1] += w1
    return m


def _host_mats():
    dense_idx = (np.arange(H * H) // H) * WP + (np.arange(H * H) % H) + 1
    pm = np.zeros((SP, PTOT), np.float32)
    um = np.zeros((PTOT, SP), np.float32)
    for s in SCALES:
        ph = _avg_mat(H, s)
        P = np.kron(ph, ph)
        uh = _bil_mat(s, H)
        U = np.kron(uh.T, uh.T)
        wps = s + 2
        for ty in range(s):
            for tx in range(s):
                r = POFF[s] + ty * wps + tx + 1
                pm[dense_idx, r] = P[ty * s + tx]
                um[r, dense_idx] = U[ty * s + tx]
    cmask = np.zeros((8, SP), np.float32)
    cmask[:, dense_idx] = 1.0
    tmask = np.zeros((8, PTOT), np.float32)
    for s in SCALES:
        wps = s + 2
        for ty in range(s):
            tmask[:, POFF[s] + ty * wps + 1: POFF[s] + ty * wps + 1 + s] = 1.0
    return pm, um, cmask, tmask


_PM_NP, _UM_NP, _CMASK_NP, _TMASK_NP = _host_mats()


def _f3(w):
    cout, cin = w.shape[0], w.shape[1]
    return jnp.transpose(w, (0, 2, 3, 1)).reshape(cout, 9 * cin).astype(BF16)


def _col(b):
    return b.reshape(-1, 1).astype(F32)


def _prep_db(w0, b0, w1, b1, w2, b2, w3, b3, w11, b11):
    g0 = jnp.concatenate(
        [_f3(w0), _f3(w1[:, :32]), _f3(w2[:, :32]), _f3(w3[:, :32])], axis=0)
    g1 = jnp.concatenate(
        [_f3(w1[:, 32:48]), _f3(w2[:, 32:48]), _f3(w3[:, 32:48])], axis=0)
    g2 = jnp.concatenate([_f3(w2[:, 48:64]), _f3(w3[:, 48:64])], axis=0)
    g3 = _f3(w3[:, 64:80])
    bcat = jnp.concatenate([_col(b0), _col(b1), _col(b2), _col(b3)], axis=0)
    return (g0, g1, g2, g3, bcat,
            w11.reshape(32, 96).astype(BF16), _col(b11))


def _body(x_ref, cm_ref, tm_ref, pm_ref, um_ref, win_ref, bin_ref,
          d1g0, d1g1, d1g2, d1g3, d1b, d1w11, d1b11,
          d2g0, d2g1, d2g2, d2g3, d2b, d2w11, d2b11,
          d3g0, d3g1, d3g2, d3g3, d3b, d3w11, d3b11,
          gffw_ref, gffb_ref,
          c0w1, c0b1, c0w2, c0b2,
          p2w1, p2b1, p2w2, p2b2, p2rb,
          p4w1, p4b1, p4w2, p4b2, p4rb,
          p8w1, p8b1, p8w2, p8b2, p8rb,
          p16w1, p16b1, p16w2, p16b2, p16rb,
          rbb_ref, wout_ref, bout_ref,
          out_ref,
          featp, patch, acc4, dcat, s2buf, pooled, bpcat, tb, tbh):
    cm = cm_ref[0:1, :]

    featp[:, 0:HALO] = jnp.zeros((96, HALO), BF16)
    featp[:, HALO + SP:] = jnp.zeros((96, BUFW - HALO - SP), BF16)

    def mkpatch(r0, cin):
        for dy in range(3):
            for dx in range(3):
                t = dy * 3 + dx
                off = dy * WP + dx
                patch[t * cin:(t + 1) * cin, :] = featp[r0:r0 + cin,
                                                        off:off + SP]

    def dot(a, b):
        return jnp.dot(a, b, preferred_element_type=F32)

    featp[0:1, HALO:HALO + SP] = x_ref[...]
    mkpatch(0, 1)
    s1 = (dot(win_ref[...], patch[0:9, :]) + bin_ref[...]) * cm
    featp[0:32, HALO:HALO + SP] = s1.astype(BF16)

    def dense_block(g0, g1, g2, g3, bc, w11, b11):
        mkpatch(0, 32)
        acc4[...] = dot(g0[...], patch[0:288, :])
        for i in range(1, 5):
            o = jnp.maximum(acc4[16 * (i - 1):16 * i, :]
                            + bc[16 * (i - 1):16 * i, :], 0.0) * cm
            featp[16 * i + 16:16 * i + 32, HALO:HALO + SP] = o.astype(BF16)
            if i < 4:
                mkpatch(16 * i + 16, 16)
                g = (g1, g2, g3)[i - 1]
                acc4[16 * i:64, :] += dot(g[...], patch[0:144, :])
        return (dot(w11[...], featp[0:96, HALO:HALO + SP])
                + b11[...]) * cm

    dbs = ((d1g0, d1g1, d1g2, d1g3, d1b, d1w11, d1b11),
           (d2g0, d2g1, d2g2, d2g3, d2b, d2w11, d2b11),
           (d3g0, d3g1, d3g2, d3g3, d3b, d3w11, d3b11))
    for k, refs in enumerate(dbs):
        d = dense_block(*refs)
        dcat[32 * k:32 * k + 32, :] = d.astype(BF16)
        featp[0:32, HALO:HALO + SP] = d.astype(BF16)

    s2 = (dot(gffw_ref[...], dcat[...]) + gffb_ref[...]) * cm
    s2buf[...] = s2

    pooled[...] = dot(s2.astype(BF16), pm_ref[...])

    pyr = {2: (p2w1, p2b1, p2w2, p2b2, p2rb),
           4: (p4w1, p4b1, p4w2, p4b2, p4rb),
           8: (p8w1, p8b1, p8w2, p8b2, p8rb),
           16: (p16w1, p16b1, p16w2, p16b2, p16rb)}
    for s in SCALES:
        w1r, b1r, w2r, b2r, rbr = pyr[s]
        off = POFF[s]
        wps = s + 2
        ps = s * wps
        hs = wps + 1
        tmsk = tm_ref[0:1, off:off + ps]
        tb[...] = jnp.zeros_like(tb)
        tbh[...] = jnp.zeros_like(tbh)
        tb[:, hs:hs + ps] = pooled[:, off:off + ps].astype(BF16)
        for dy in range(3):
            for dx in range(3):
                t = dy * 3 + dx
                o2 = dy * wps + dx
                patch[t * 32:(t + 1) * 32, 0:ps] = tb[0:32, o2:o2 + ps]
        h1 = jnp.maximum(dot(w1r[...], patch[0:288, 0:ps]) + b1r[...],
                         0.0) * tmsk
        tbh[:, hs:hs + ps] = h1.astype(BF16)
        for dy in range(3):
            for dx in range(3):
                t = dy * 3 + dx
                o2 = dy * wps + dx
                patch[t * 32:(t + 1) * 32, 0:ps] = tbh[0:32, o2:o2 + ps]
        h2 = jnp.maximum(dot(w2r[...], patch[0:288, 0:ps]) + b2r[...],
                         0.0) * tmsk
        bout = pooled[:, off:off + ps] + h2
        bp = dot(rbr[...], bout.astype(BF16))
        bpcat[:, off:off + ps] = bp.astype(BF16)

    uu = dot(bpcat[...], um_ref[...])

    s3 = (s2buf[...] + uu + rbb_ref[...]) * cm
    s2buf[...] = s3
    featp[0:32, HALO:HALO + SP] = s3.astype(BF16)
    mkpatch(0, 32)
    h1 = jnp.maximum(dot(c0w1[...], patch[0:288, :]) + c0b1[...], 0.0) * cm
    featp[32:64, HALO:HALO + SP] = h1.astype(BF16)
    mkpatch(32, 32)
    h2 = jnp.maximum(dot(c0w2[...], patch[0:288, :]) + c0b2[...], 0.0) * cm
    s3b = s2buf[...] * 2.0 + h2
    featp[0:32, HALO:HALO + SP] = s3b.astype(BF16)
    mkpatch(0, 32)
    out = dot(wout_ref[...], patch[0:288, :])[0:1, :] + bout_ref[0:1, :]
    out_ref[...] = out


def kernel(x, conv_in_w, conv_in_b,
           db1_dense0_w, db1_dense0_b, db1_dense1_w, db1_dense1_b,
           db1_dense2_w, db1_dense2_b, db1_dense3_w, db1_dense3_b,
           db1_1x1_w, db1_1x1_b,
           db2_dense0_w, db2_dense0_b, db2_dense1_w, db2_dense1_b,
           db2_dense2_w, db2_dense2_b, db2_dense3_w, db2_dense3_b,
           db2_1x1_w, db2_1x1_b,
           db3_dense0_w, db3_dense0_b, db3_dense1_w, db3_dense1_b,
           db3_dense2_w, db3_dense2_b, db3_dense3_w, db3_dense3_b,
           db3_1x1_w, db3_1x1_b,
           gff_1x1_w, gff_1x1_b,
           bc0_0_w, bc0_0_b, bc0_1_w, bc0_1_b,
           bc2_0_w, bc2_0_b, bc2_1_w, bc2_1_b,
           bc4_0_w, bc4_0_b, bc4_1_w, bc4_1_b,
           bc8_0_w, bc8_0_b, bc8_1_w, bc8_1_b,
           bc16_0_w, bc16_0_b, bc16_1_w, bc16_1_b,
           rb_w, rb_b,
           conv_out_w, conv_out_b):
    N = x.shape[0]

    xp = jnp.pad(x, ((0, 0), (0, 0), (0, 0), (1, 1)))
    xp = xp.reshape(N, 1, SP).astype(BF16)

    pm = jnp.asarray(_PM_NP, BF16)
    um = jnp.asarray(_UM_NP, BF16)
    cmask = jnp.asarray(_CMASK_NP)
    tmask = jnp.asarray(_TMASK_NP)

    db1 = _prep_db(db1_dense0_w, db1_dense0_b, db1_dense1_w, db1_dense1_b,
                   db1_dense2_w, db1_dense2_b, db1_dense3_w, db1_dense3_b,
                   db1_1x1_w, db1_1x1_b)
    db2 = _prep_db(db2_dense0_w, db2_dense0_b, db2_dense1_w, db2_dense1_b,
                   db2_dense2_w, db2_dense2_b, db2_dense3_w, db2_dense3_b,
                   db2_1x1_w, db2_1x1_b)
    db3 = _prep_db(db3_dense0_w, db3_dense0_b, db3_dense1_w, db3_dense1_b,
                   db3_dense2_w, db3_dense2_b, db3_dense3_w, db3_dense3_b,
                   db3_1x1_w, db3_1x1_b)

    rbw = rb_w.reshape(32, 128).astype(BF16)
    pyr_args = []
    for i, s in enumerate(SCALES):
        w1, b1, w2, b2 = {2: (bc2_0_w, bc2_0_b, bc2_1_w, bc2_1_b),
                          4: (bc4_0_w, bc4_0_b, bc4_1_w, bc4_1_b),
                          8: (bc8_0_w, bc8_0_b, bc8_1_w, bc8_1_b),
                          16: (bc16_0_w, bc16_0_b, bc16_1_w, bc16_1_b)}[s]
        pyr_args += [_f3(w1), _col(b1), _f3(w2), _col(b2),
                     rbw[:, 32 * i:32 * i + 32]]

    wout = jnp.zeros((8, 288), BF16).at[0:1, :].set(_f3(conv_out_w))
    bout = jnp.zeros((8, 1), F32).at[0, 0].set(conv_out_b[0])

    operands = [xp, cmask, tmask, pm, um,
                _f3(conv_in_w), _col(conv_in_b),
                *db1, *db2, *db3,
                gff_1x1_w.reshape(32, 96).astype(BF16), _col(gff_1x1_b),
                _f3(bc0_0_w), _col(bc0_0_b), _f3(bc0_1_w), _col(bc0_1_b),
                *pyr_args,
                _col(rb_b), wout, bout]

    grid = (N,)

    def xmap(n):
        return (n, 0, 0)

    def wmap(n):
        return (0, 0)

    in_specs = [pl.BlockSpec((None, 1, SP), xmap)]
    in_specs += [pl.BlockSpec(op.shape, wmap) for op in operands[1:]]

    out = pl.pallas_call(
        _body,
        out_shape=jax.ShapeDtypeStruct((N, 1, SP), F32),
        grid=grid,
        in_specs=in_specs,
        out_specs=pl.BlockSpec((None, 1, SP), xmap),
        scratch_shapes=[
            pltpu.VMEM((96, BUFW), BF16),
            pltpu.VMEM((288, SP), BF16),
            pltpu.VMEM((64, SP), F32),
            pltpu.VMEM((96, SP), BF16),
            pltpu.VMEM((32, SP), F32),
            pltpu.VMEM((32, PTOT), F32),
            pltpu.VMEM((32, PTOT), BF16),
            pltpu.VMEM((32, 326), BF16),
            pltpu.VMEM((32, 326), BF16),
        ],
        compiler_params=pltpu.CompilerParams(
            dimension_semantics=("parallel",)),
    )(*operands)

    out = out.reshape(N, H, WP)[:, :, 1:H + 1]
    return out.reshape(N, 1, H, H)


# speedup vs baseline: 1.0154x; 1.0154x over previous
import numpy as np

import jax
import jax.numpy as jnp
from jax.experimental import pallas as pl
from jax.experimental.pallas import tpu as pltpu

F32 = jnp.float32
BF16 = jnp.bfloat16

H = 64
WP = H + 2
SP = H * WP
HALO = WP + 1
BUFW = SP + 2 * HALO
SCALES = (2, 4, 8, 16)
POFF = {2: 0, 4: 8, 8: 32, 16: 112}
PTOT = 400
NB = 2


def _avg_mat(in_size, out_size):
    m = np.zeros((out_size, in_size), np.float32)
    for i in range(out_size):
        start = (i * in_size) // out_size
        end = -(-((i + 1) * in_size) // out_size)
        m[i, start:end] = 1.0 / (end - start)
    return m


def _bil_mat(in_size, out_size):
    m = np.zeros((out_size, in_size), np.float32)
    if out_size == 1 or in_size == 1:
        m[:, 0] = 1.0
        return m
    scale = (in_size - 1) / (out_size - 1)
    for i in range(out_size):
        src = i * scale
        i0 = min(int(np.floor(src)), in_size - 1)
        i1 = min(i0 + 1, in_size - 1)
        w1 = src - i0
        m[i, i0] += 1.0 - w1
        m[i, i1] += w1
    return m


def _host_mats():
    dense_idx = (np.arange(H * H) // H) * WP + (np.arange(H * H) % H) + 1
    pm = np.zeros((SP, PTOT), np.float32)
    um = np.zeros((PTOT, SP), np.float32)
    for s in SCALES:
        ph = _avg_mat(H, s)
        P = np.kron(ph, ph)
        uh = _bil_mat(s, H)
        U = np.kron(uh.T, uh.T)
        wps = s + 2
        for ty in range(s):
            for tx in range(s):
                r = POFF[s] + ty * wps + tx + 1
                pm[dense_idx, r] = P[ty * s + tx]
                um[r, dense_idx] = U[ty * s + tx]
    cmask = np.zeros((8, SP), np.float32)
    cmask[:, dense_idx] = 1.0
    tmask = np.zeros((8, PTOT), np.float32)
    for s in SCALES:
        wps = s + 2
        for ty in range(s):
            tmask[:, POFF[s] + ty * wps + 1: POFF[s] + ty * wps + 1 + s] = 1.0
    return pm, um, cmask, tmask


_PM_NP, _UM_NP, _CMASK_NP, _TMASK_NP = _host_mats()


def _f3(w):
    cout, cin = w.shape[0], w.shape[1]
    return jnp.transpose(w, (0, 2, 3, 1)).reshape(cout, 9 * cin).astype(BF16)


def _col(b):
    return b.reshape(-1, 1).astype(F32)


def _prep_db(w0, b0, w1, b1, w2, b2, w3, b3, w11, b11):
    g0 = jnp.concatenate(
        [_f3(w0), _f3(w1[:, :32]), _f3(w2[:, :32]), _f3(w3[:, :32])], axis=0)
    g1 = jnp.concatenate(
        [_f3(w1[:, 32:48]), _f3(w2[:, 32:48]), _f3(w3[:, 32:48])], axis=0)
    g2 = jnp.concatenate([_f3(w2[:, 48:64]), _f3(w3[:, 48:64])], axis=0)
    g3 = _f3(w3[:, 64:80])
    bcat = jnp.concatenate([_col(b0), _col(b1), _col(b2), _col(b3)], axis=0)
    return (g0, g1, g2, g3, bcat,
            w11.reshape(32, 96).astype(BF16), _col(b11))


def _body(x_ref, cm_ref, tm_ref, pm_ref, um_ref, win_ref, bin_ref,
          d1g0, d1g1, d1g2, d1g3, d1b, d1w11, d1b11,
          d2g0, d2g1, d2g2, d2g3, d2b, d2w11, d2b11,
          d3g0, d3g1, d3g2, d3g3, d3b, d3w11, d3b11,
          gffw_ref, gffb_ref,
          c0w1, c0b1, c0w2, c0b2,
          p2w1, p2b1, p2w2, p2b2, p2rb,
          p4w1, p4b1, p4w2, p4b2, p4rb,
          p8w1, p8b1, p8w2, p8b2, p8rb,
          p16w1, p16b1, p16w2, p16b2, p16rb,
          rbb_ref, wout_ref, bout_ref,
          out_ref,
          featp0, featp1, patch0, patch1, acc40, acc41, dcat0, dcat1,
          s2buf0, s2buf1, pooled0, pooled1, bpcat0, bpcat1,
          tb0, tb1, tbh0, tbh1):
    cm = cm_ref[0:1, :]
    dbs = ((d1g0, d1g1, d1g2, d1g3, d1b, d1w11, d1b11),
           (d2g0, d2g1, d2g2, d2g3, d2b, d2w11, d2b11),
           (d3g0, d3g1, d3g2, d3g3, d3b, d3w11, d3b11))
    pyr = {2: (p2w1, p2b1, p2w2, p2b2, p2rb),
           4: (p4w1, p4b1, p4w2, p4b2, p4rb),
           8: (p8w1, p8b1, p8w2, p8b2, p8rb),
           16: (p16w1, p16b1, p16w2, p16b2, p16rb)}

    def dot(a, b):
        return jnp.dot(a, b, preferred_element_type=F32)

    def stages_for(slot):
        featp = (featp0, featp1)[slot]
        patch = (patch0, patch1)[slot]
        acc4 = (acc40, acc41)[slot]
        dcat = (dcat0, dcat1)[slot]
        s2buf = (s2buf0, s2buf1)[slot]
        pooled = (pooled0, pooled1)[slot]
        bpcat = (bpcat0, bpcat1)[slot]
        tb = (tb0, tb1)[slot]
        tbh = (tbh0, tbh1)[slot]
        F = P = A = D = S = 0

        def mkpatch(r0, cin):
            for dy in range(3):
                for dx in range(3):
                    t = dy * 3 + dx
                    off = dy * WP + dx
                    patch[P + t * cin:P + (t + 1) * cin, :] = (
                        featp[F + r0:F + r0 + cin, off:off + SP])

        st = []

        def s_conv_in():
            featp[F:F + 96, 0:HALO] = jnp.zeros((96, HALO), BF16)
            featp[F:F + 96, HALO + SP:] = jnp.zeros((96, BUFW - HALO - SP),
                                                    BF16)
            featp[F:F + 1, HALO:HALO + SP] = x_ref[slot:slot + 1, :]
            mkpatch(0, 1)
            s1 = (dot(win_ref[...], patch[P:P + 9, :]) + bin_ref[...]) * cm
            featp[F:F + 32, HALO:HALO + SP] = s1.astype(BF16)
        st.append(s_conv_in)

        for k in range(3):
            def s_db_g0(k=k):
                g0 = dbs[k][0]
                mkpatch(0, 32)
                acc4[A:A + 64, :] = dot(g0[...], patch[P:P + 288, :])
            st.append(s_db_g0)

            for i in range(1, 4):
                def s_db_gi(k=k, i=i):
                    g = dbs[k][i]
                    bc = dbs[k][4]
                    o = jnp.maximum(acc4[A + 16 * (i - 1):A + 16 * i, :]
                                    + bc[16 * (i - 1):16 * i, :], 0.0) * cm
                    featp[F + 16 * i + 16:F + 16 * i + 32,
                          HALO:HALO + SP] = o.astype(BF16)
                    mkpatch(16 * i + 16, 16)
                    acc4[A + 16 * i:A + 64, :] += dot(g[...],
                                                      patch[P:P + 144, :])
                st.append(s_db_gi)

            def s_db_end(k=k):
                bc, w11, b11 = dbs[k][4], dbs[k][5], dbs[k][6]
                o = jnp.maximum(acc4[A + 48:A + 64, :] + bc[48:64, :],
                                0.0) * cm
                featp[F + 80:F + 96, HALO:HALO + SP] = o.astype(BF16)
                d = (dot(w11[...], featp[F:F + 96, HALO:HALO + SP])
                     + b11[...]) * cm
                dcat[D + 32 * k:D + 32 * k + 32, :] = d.astype(BF16)
                featp[F:F + 32, HALO:HALO + SP] = d.astype(BF16)
            st.append(s_db_end)

        def s_gff_pool():
            s2 = (dot(gffw_ref[...], dcat[D:D + 96, :]) + gffb_ref[...]) * cm
            s2buf[S:S + 32, :] = s2
            pooled[S:S + 32, :] = dot(s2.astype(BF16), pm_ref[...])
        st.append(s_gff_pool)

        for s in SCALES:
            def s_pyr(s=s):
                w1r, b1r, w2r, b2r, rbr = pyr[s]
                off = POFF[s]
                wps = s + 2
                ps = s * wps
                hs = wps + 1
                tmsk = tm_ref[0:1, off:off + ps]
                tb[S:S + 32, :] = jnp.zeros((32, 326), BF16)
                tbh[S:S + 32, :] = jnp.zeros((32, 326), BF16)
                tb[S:S + 32, hs:hs + ps] = (
                    pooled[S:S + 32, off:off + ps].astype(BF16))
                for dy in range(3):
                    for dx in range(3):
                        t = dy * 3 + dx
                        o2 = dy * wps + dx
                        patch[P + t * 32:P + (t + 1) * 32, 0:ps] = (
                            tb[S:S + 32, o2:o2 + ps])
                h1 = jnp.maximum(
                    dot(w1r[...], patch[P:P + 288, 0:ps]) + b1r[...],
                    0.0) * tmsk
                tbh[S:S + 32, hs:hs + ps] = h1.astype(BF16)
                for dy in range(3):
                    for dx in range(3):
                        t = dy * 3 + dx
                        o2 = dy * wps + dx
                        patch[P + t * 32:P + (t + 1) * 32, 0:ps] = (
                            tbh[S:S + 32, o2:o2 + ps])
                h2 = jnp.maximum(
                    dot(w2r[...], patch[P:P + 288, 0:ps]) + b2r[...],
                    0.0) * tmsk
                bout = pooled[S:S + 32, off:off + ps] + h2
                bp = dot(rbr[...], bout.astype(BF16))
                bpcat[S:S + 32, off:off + ps] = bp.astype(BF16)
            st.append(s_pyr)

        def s_up_s3():
            uu = dot(bpcat[S:S + 32, :], um_ref[...])
            s3 = (s2buf[S:S + 32, :] + uu + rbb_ref[...]) * cm
            s2buf[S:S + 32, :] = s3
            featp[F:F + 32, HALO:HALO + SP] = s3.astype(BF16)
        st.append(s_up_s3)

        def s_bc0_1():
            mkpatch(0, 32)
            h1 = jnp.maximum(dot(c0w1[...], patch[P:P + 288, :])
                             + c0b1[...], 0.0) * cm
            featp[F + 32:F + 64, HALO:HALO + SP] = h1.astype(BF16)
        st.append(s_bc0_1)

        def s_bc0_2():
            mkpatch(32, 32)
            h2 = jnp.maximum(dot(c0w2[...], patch[P:P + 288, :])
                             + c0b2[...], 0.0) * cm
            s3b = s2buf[S:S + 32, :] * 2.0 + h2
            featp[F:F + 32, HALO:HALO + SP] = s3b.astype(BF16)
        st.append(s_bc0_2)

        def s_conv_out():
            mkpatch(0, 32)
            out = (dot(wout_ref[...], patch[P:P + 288, :])[0:1, :]
                   + bout_ref[0:1, :])
            out_ref[slot:slot + 1, :] = out
        st.append(s_conv_out)

        return st

    for fa, fb in zip(stages_for(0), stages_for(1)):
        fa()
        fb()


def kernel(x, conv_in_w, conv_in_b,
           db1_dense0_w, db1_dense0_b, db1_dense1_w, db1_dense1_b,
           db1_dense2_w, db1_dense2_b, db1_dense3_w, db1_dense3_b,
           db1_1x1_w, db1_1x1_b,
           db2_dense0_w, db2_dense0_b, db2_dense1_w, db2_dense1_b,
           db2_dense2_w, db2_dense2_b, db2_dense3_w, db2_dense3_b,
           db2_1x1_w, db2_1x1_b,
           db3_dense0_w, db3_dense0_b, db3_dense1_w, db3_dense1_b,
           db3_dense2_w, db3_dense2_b, db3_dense3_w, db3_dense3_b,
           db3_1x1_w, db3_1x1_b,
           gff_1x1_w, gff_1x1_b,
           bc0_0_w, bc0_0_b, bc0_1_w, bc0_1_b,
           bc2_0_w, bc2_0_b, bc2_1_w, bc2_1_b,
           bc4_0_w, bc4_0_b, bc4_1_w, bc4_1_b,
           bc8_0_w, bc8_0_b, bc8_1_w, bc8_1_b,
           bc16_0_w, bc16_0_b, bc16_1_w, bc16_1_b,
           rb_w, rb_b,
           conv_out_w, conv_out_b):
    N = x.shape[0]

    xp = jnp.pad(x, ((0, 0), (0, 0), (0, 0), (1, 1)))
    xp = xp.reshape(N // NB, NB, SP).astype(BF16)

    pm = jnp.asarray(_PM_NP, BF16)
    um = jnp.asarray(_UM_NP, BF16)
    cmask = jnp.asarray(_CMASK_NP)
    tmask = jnp.asarray(_TMASK_NP)

    db1 = _prep_db(db1_dense0_w, db1_dense0_b, db1_dense1_w, db1_dense1_b,
                   db1_dense2_w, db1_dense2_b, db1_dense3_w, db1_dense3_b,
                   db1_1x1_w, db1_1x1_b)
    db2 = _prep_db(db2_dense0_w, db2_dense0_b, db2_dense1_w, db2_dense1_b,
                   db2_dense2_w, db2_dense2_b, db2_dense3_w, db2_dense3_b,
                   db2_1x1_w, db2_1x1_b)
    db3 = _prep_db(db3_dense0_w, db3_dense0_b, db3_dense1_w, db3_dense1_b,
                   db3_dense2_w, db3_dense2_b, db3_dense3_w, db3_dense3_b,
                   db3_1x1_w, db3_1x1_b)

    rbw = rb_w.reshape(32, 128).astype(BF16)
    pyr_args = []
    for i, s in enumerate(SCALES):
        w1, b1, w2, b2 = {2: (bc2_0_w, bc2_0_b, bc2_1_w, bc2_1_b),
                          4: (bc4_0_w, bc4_0_b, bc4_1_w, bc4_1_b),
                          8: (bc8_0_w, bc8_0_b, bc8_1_w, bc8_1_b),
                          16: (bc16_0_w, bc16_0_b, bc16_1_w, bc16_1_b)}[s]
        pyr_args += [_f3(w1), _col(b1), _f3(w2), _col(b2),
                     rbw[:, 32 * i:32 * i + 32]]

    wout = jnp.zeros((8, 288), BF16).at[0:1, :].set(_f3(conv_out_w))
    bout = jnp.zeros((8, 1), F32).at[0, 0].set(conv_out_b[0])

    operands = [xp, cmask, tmask, pm, um,
                _f3(conv_in_w), _col(conv_in_b),
                *db1, *db2, *db3,
                gff_1x1_w.reshape(32, 96).astype(BF16), _col(gff_1x1_b),
                _f3(bc0_0_w), _col(bc0_0_b), _f3(bc0_1_w), _col(bc0_1_b),
                *pyr_args,
                _col(rb_b), wout, bout]

    grid = (N // NB,)

    def xmap(n):
        return (n, 0, 0)

    def wmap(n):
        return (0, 0)

    in_specs = [pl.BlockSpec((None, NB, SP), xmap)]
    in_specs += [pl.BlockSpec(op.shape, wmap) for op in operands[1:]]

    out = pl.pallas_call(
        _body,
        out_shape=jax.ShapeDtypeStruct((N // NB, NB, SP), F32),
        grid=grid,
        in_specs=in_specs,
        out_specs=pl.BlockSpec((None, NB, SP), xmap),
        scratch_shapes=(
            [pltpu.VMEM((96, BUFW), BF16)] * 2
            + [pltpu.VMEM((288, SP), BF16)] * 2
            + [pltpu.VMEM((64, SP), F32)] * 2
            + [pltpu.VMEM((96, SP), BF16)] * 2
            + [pltpu.VMEM((32, SP), F32)] * 2
            + [pltpu.VMEM((32, PTOT), F32)] * 2
            + [pltpu.VMEM((32, PTOT), BF16)] * 2
            + [pltpu.VMEM((32, 326), BF16)] * 4
        ),
        compiler_params=pltpu.CompilerParams(
            dimension_semantics=("parallel",)),
    )(*operands)

    out = out.reshape(N, H, WP)[:, :, 1:H + 1]
    return out.reshape(N, 1, H, H)


# speedup vs baseline: 1.0905x; 1.0740x over previous
import numpy as np

import jax
import jax.numpy as jnp
from jax.experimental import pallas as pl
from jax.experimental.pallas import tpu as pltpu

F32 = jnp.float32
BF16 = jnp.bfloat16

H = 64
WP = H + 2
SP = H * WP
HALO = WP + 1
BUFW = SP + 2 * HALO
SCALES = (2, 4, 8, 16)
POFF = {2: 0, 4: 8, 8: 32, 16: 112}
PTOT = 400
NB = 2


def _avg_mat(in_size, out_size):
    m = np.zeros((out_size, in_size), np.float32)
    for i in range(out_size):
        start = (i * in_size) // out_size
        end = -(-((i + 1) * in_size) // out_size)
        m[i, start:end] = 1.0 / (end - start)
    return m


def _bil_mat(in_size, out_size):
    m = np.zeros((out_size, in_size), np.float32)
    if out_size == 1 or in_size == 1:
        m[:, 0] = 1.0
        return m
    scale = (in_size - 1) / (out_size - 1)
    for i in range(out_size):
        src = i * scale
        i0 = min(int(np.floor(src)), in_size - 1)
        i1 = min(i0 + 1, in_size - 1)
        w1 = src - i0
        m[i, i0] += 1.0 - w1
        m[i, i1] += w1
    return m


def _host_mats():
    dense_idx = (np.arange(H * H) // H) * WP + (np.arange(H * H) % H) + 1
    pm = np.zeros((SP, PTOT), np.float32)
    um = np.zeros((PTOT, SP), np.float32)
    for s in SCALES:
        ph = _avg_mat(H, s)
        P = np.kron(ph, ph)
        uh = _bil_mat(s, H)
        U = np.kron(uh.T, uh.T)
        wps = s + 2
        for ty in range(s):
            for tx in range(s):
                r = POFF[s] + ty * wps + tx + 1
                pm[dense_idx, r] = P[ty * s + tx]
                um[r, dense_idx] = U[ty * s + tx]
    cmask = np.zeros((8, SP), np.float32)
    cmask[:, dense_idx] = 1.0
    tmbd = np.zeros((128, PTOT), np.float32)
    for i, s in enumerate(SCALES):
        wps = s + 2
        for ty in range(s):
            tmbd[32 * i:32 * i + 32,
                 POFF[s] + ty * wps + 1: POFF[s] + ty * wps + 1 + s] = 1.0
    return pm, um, cmask, tmbd


_PM_NP, _UM_NP, _CMASK_NP, _TMBD_NP = _host_mats()

_TBASE = {2: 19, 4: 46, 8: 89, 16: 188}
TBW = 512


def _f3(w):
    cout, cin = w.shape[0], w.shape[1]
    return jnp.transpose(w, (0, 2, 3, 1)).reshape(cout, 9 * cin).astype(BF16)


def _col(b):
    return b.reshape(-1, 1).astype(F32)


def _prep_db(w0, b0, w1, b1, w2, b2, w3, b3, w11, b11):
    g0 = jnp.concatenate(
        [_f3(w0), _f3(w1[:, :32]), _f3(w2[:, :32]), _f3(w3[:, :32])], axis=0)
    g1 = jnp.concatenate(
        [_f3(w1[:, 32:48]), _f3(w2[:, 32:48]), _f3(w3[:, 32:48])], axis=0)
    g2 = jnp.concatenate([_f3(w2[:, 48:64]), _f3(w3[:, 48:64])], axis=0)
    g3 = _f3(w3[:, 64:80])
    bcat = jnp.concatenate([_col(b0), _col(b1), _col(b2), _col(b3)], axis=0)
    return (g0, g1, g2, g3, bcat,
            w11.reshape(32, 96).astype(BF16), _col(b11))


def _body(x_ref, cm_ref, tm_ref, pm_ref, um_ref, win_ref, bin_ref,
          d1g0, d1g1, d1g2, d1g3, d1b, d1w11, d1b11,
          d2g0, d2g1, d2g2, d2g3, d2b, d2w11, d2b11,
          d3g0, d3g1, d3g2, d3g3, d3b, d3w11, d3b11,
          gffw_ref, gffb_ref,
          c0w1, c0b1, c0w2, c0b2,
          w1bd_ref, b1bd_ref, w2bd_ref, b2bd_ref, rbcat_ref,
          rbb_ref, wout_ref, bout_ref,
          out_ref,
          featp0, featp1, patch0, patch1, acc40, acc41, dcat0, dcat1,
          s2buf0, s2buf1, pooled0, pooled1, bpcat0, bpcat1,
          pbd0, pbd1, tbbd0, tbbd1, tbhbd0, tbhbd1, boutbd0, boutbd1):
    cm = cm_ref[0:1, :]
    dbs = ((d1g0, d1g1, d1g2, d1g3, d1b, d1w11, d1b11),
           (d2g0, d2g1, d2g2, d2g3, d2b, d2w11, d2b11),
           (d3g0, d3g1, d3g2, d3g3, d3b, d3w11, d3b11))

    def dot(a, b):
        return jnp.dot(a, b, preferred_element_type=F32)

    def stages_for(slot):
        featp = (featp0, featp1)[slot]
        patch = (patch0, patch1)[slot]
        acc4 = (acc40, acc41)[slot]
        dcat = (dcat0, dcat1)[slot]
        s2buf = (s2buf0, s2buf1)[slot]
        pooled = (pooled0, pooled1)[slot]
        bpcat = (bpcat0, bpcat1)[slot]
        pbd = (pbd0, pbd1)[slot]
        tbbd = (tbbd0, tbbd1)[slot]
        tbhbd = (tbhbd0, tbhbd1)[slot]
        boutbd = (boutbd0, boutbd1)[slot]
        F = P = A = D = S = 0

        def mkpatch(r0, cin):
            for dy in range(3):
                for dx in range(3):
                    t = dy * 3 + dx
                    off = dy * WP + dx
                    patch[P + t * cin:P + (t + 1) * cin, :] = (
                        featp[F + r0:F + r0 + cin, off:off + SP])

        st = []

        def s_conv_in():
            featp[F:F + 96, 0:HALO] = jnp.zeros((96, HALO), BF16)
            featp[F:F + 96, HALO + SP:] = jnp.zeros((96, BUFW - HALO - SP),
                                                    BF16)
            featp[F:F + 1, HALO:HALO + SP] = x_ref[slot:slot + 1, :]
            mkpatch(0, 1)
            s1 = (dot(win_ref[...], patch[P:P + 9, :]) + bin_ref[...]) * cm
            featp[F:F + 32, HALO:HALO + SP] = s1.astype(BF16)
        st.append(s_conv_in)

        for k in range(3):
            def s_db_g0(k=k):
                g0 = dbs[k][0]
                mkpatch(0, 32)
                acc4[A:A + 64, :] = dot(g0[...], patch[P:P + 288, :])
            st.append(s_db_g0)

            for i in range(1, 4):
                def s_db_gi(k=k, i=i):
                    g = dbs[k][i]
                    bc = dbs[k][4]
                    o = jnp.maximum(acc4[A + 16 * (i - 1):A + 16 * i, :]
                                    + bc[16 * (i - 1):16 * i, :], 0.0) * cm
                    featp[F + 16 * i + 16:F + 16 * i + 32,
                          HALO:HALO + SP] = o.astype(BF16)
                    mkpatch(16 * i + 16, 16)
                    acc4[A + 16 * i:A + 64, :] += dot(g[...],
                                                      patch[P:P + 144, :])
                st.append(s_db_gi)

            def s_db_end(k=k):
                bc, w11, b11 = dbs[k][4], dbs[k][5], dbs[k][6]
                o = jnp.maximum(acc4[A + 48:A + 64, :] + bc[48:64, :],
                                0.0) * cm
                featp[F + 80:F + 96, HALO:HALO + SP] = o.astype(BF16)
                d = (dot(w11[...], featp[F:F + 96, HALO:HALO + SP])
                     + b11[...]) * cm
                dcat[D + 32 * k:D + 32 * k + 32, :] = d.astype(BF16)
                featp[F:F + 32, HALO:HALO + SP] = d.astype(BF16)
            st.append(s_db_end)

        def s_gff_pool():
            s2 = (dot(gffw_ref[...], dcat[D:D + 96, :]) + gffb_ref[...]) * cm
            s2buf[S:S + 32, :] = s2
            pooled[S:S + 32, :] = dot(s2.astype(BF16), pm_ref[...])
        st.append(s_gff_pool)

        def pbd_fill(src):
            for i, s in enumerate(SCALES):
                off = POFF[s]
                wps = s + 2
                ps = s * wps
                for dy in range(3):
                    for dx in range(3):
                        t = dy * 3 + dx
                        o2 = _TBASE[s] + (dy - 1) * wps + (dx - 1)
                        pbd[288 * i + t * 32:288 * i + (t + 1) * 32,
                            off:off + ps] = src[32 * i:32 * i + 32,
                                                o2:o2 + ps]

        def s_pyr_conv1():
            pbd[...] = jnp.zeros_like(pbd)
            tbbd[...] = jnp.zeros_like(tbbd)
            for i, s in enumerate(SCALES):
                ps = s * (s + 2)
                tbbd[32 * i:32 * i + 32, _TBASE[s]:_TBASE[s] + ps] = (
                    pooled[S:S + 32, POFF[s]:POFF[s] + ps].astype(BF16))
            pbd_fill(tbbd)
            h1 = jnp.maximum(dot(w1bd_ref[...], pbd[...]) + b1bd_ref[...],
                             0.0) * tm_ref[...]
            tbhbd[...] = jnp.zeros_like(tbhbd)
            for i, s in enumerate(SCALES):
                ps = s * (s + 2)
                tbhbd[32 * i:32 * i + 32, _TBASE[s]:_TBASE[s] + ps] = (
                    h1[32 * i:32 * i + 32,
                       POFF[s]:POFF[s] + ps].astype(BF16))
        st.append(s_pyr_conv1)

        def s_pyr_conv2():
            pbd_fill(tbhbd)
            h2 = jnp.maximum(dot(w2bd_ref[...], pbd[...]) + b2bd_ref[...],
                             0.0) * tm_ref[...]
            boutbd[...] = jnp.zeros_like(boutbd)
            for i, s in enumerate(SCALES):
                ps = s * (s + 2)
                boutbd[32 * i:32 * i + 32, POFF[s]:POFF[s] + ps] = (
                    (pooled[S:S + 32, POFF[s]:POFF[s] + ps]
                     + h2[32 * i:32 * i + 32,
                          POFF[s]:POFF[s] + ps]).astype(BF16))
            bpcat[S:S + 32, :] = dot(rbcat_ref[...],
                                     boutbd[...]).astype(BF16)
        st.append(s_pyr_conv2)

        def s_up_s3():
            uu = dot(bpcat[S:S + 32, :], um_ref[...])
            s3 = (s2buf[S:S + 32, :] + uu + rbb_ref[...]) * cm
            s2buf[S:S + 32, :] = s3
            featp[F:F + 32, HALO:HALO + SP] = s3.astype(BF16)
        st.append(s_up_s3)

        def s_bc0_1():
            mkpatch(0, 32)
            h1 = jnp.maximum(dot(c0w1[...], patch[P:P + 288, :])
                             + c0b1[...], 0.0) * cm
            featp[F + 32:F + 64, HALO:HALO + SP] = h1.astype(BF16)
        st.append(s_bc0_1)

        def s_bc0_2():
            mkpatch(32, 32)
            h2 = jnp.maximum(dot(c0w2[...], patch[P:P + 288, :])
                             + c0b2[...], 0.0) * cm
            s3b = s2buf[S:S + 32, :] * 2.0 + h2
            featp[F:F + 32, HALO:HALO + SP] = s3b.astype(BF16)
        st.append(s_bc0_2)

        def s_conv_out():
            mkpatch(0, 32)
            out = (dot(wout_ref[...], patch[P:P + 288, :])[0:1, :]
                   + bout_ref[0:1, :])
            out_ref[slot:slot + 1, :] = out
        st.append(s_conv_out)

        return st

    for fa, fb in zip(stages_for(0), stages_for(1)):
        fa()
        fb()


def kernel(x, conv_in_w, conv_in_b,
           db1_dense0_w, db1_dense0_b, db1_dense1_w, db1_dense1_b,
           db1_dense2_w, db1_dense2_b, db1_dense3_w, db1_dense3_b,
           db1_1x1_w, db1_1x1_b,
           db2_dense0_w, db2_dense0_b, db2_dense1_w, db2_dense1_b,
           db2_dense2_w, db2_dense2_b, db2_dense3_w, db2_dense3_b,
           db2_1x1_w, db2_1x1_b,
           db3_dense0_w, db3_dense0_b, db3_dense1_w, db3_dense1_b,
           db3_dense2_w, db3_dense2_b, db3_dense3_w, db3_dense3_b,
           db3_1x1_w, db3_1x1_b,
           gff_1x1_w, gff_1x1_b,
           bc0_0_w, bc0_0_b, bc0_1_w, bc0_1_b,
           bc2_0_w, bc2_0_b, bc2_1_w, bc2_1_b,
           bc4_0_w, bc4_0_b, bc4_1_w, bc4_1_b,
           bc8_0_w, bc8_0_b, bc8_1_w, bc8_1_b,
           bc16_0_w, bc16_0_b, bc16_1_w, bc16_1_b,
           rb_w, rb_b,
           conv_out_w, conv_out_b):
    N = x.shape[0]

    xp = jnp.pad(x, ((0, 0), (0, 0), (0, 0), (1, 1)))
    xp = xp.reshape(N // NB, NB, SP).astype(BF16)

    pm = jnp.asarray(_PM_NP, BF16)
    um = jnp.asarray(_UM_NP, BF16)
    cmask = jnp.asarray(_CMASK_NP)
    tmbd = jnp.asarray(_TMBD_NP)

    db1 = _prep_db(db1_dense0_w, db1_dense0_b, db1_dense1_w, db1_dense1_b,
                   db1_dense2_w, db1_dense2_b, db1_dense3_w, db1_dense3_b,
                   db1_1x1_w, db1_1x1_b)
    db2 = _prep_db(db2_dense0_w, db2_dense0_b, db2_dense1_w, db2_dense1_b,
                   db2_dense2_w, db2_dense2_b, db2_dense3_w, db2_dense3_b,
                   db2_1x1_w, db2_1x1_b)
    db3 = _prep_db(db3_dense0_w, db3_dense0_b, db3_dense1_w, db3_dense1_b,
                   db3_dense2_w, db3_dense2_b, db3_dense3_w, db3_dense3_b,
                   db3_1x1_w, db3_1x1_b)

    rbcat = rb_w.reshape(32, 128).astype(BF16)
    w1bd = jnp.zeros((128, 1152), BF16)
    w2bd = jnp.zeros((128, 1152), BF16)
    b1bd = jnp.zeros((128, 1), F32)
    b2bd = jnp.zeros((128, 1), F32)
    for i, s in enumerate(SCALES):
        w1, b1, w2, b2 = {2: (bc2_0_w, bc2_0_b, bc2_1_w, bc2_1_b),
                          4: (bc4_0_w, bc4_0_b, bc4_1_w, bc4_1_b),
                          8: (bc8_0_w, bc8_0_b, bc8_1_w, bc8_1_b),
                          16: (bc16_0_w, bc16_0_b, bc16_1_w, bc16_1_b)}[s]
        w1bd = w1bd.at[32 * i:32 * i + 32,
                       288 * i:288 * i + 288].set(_f3(w1))
        w2bd = w2bd.at[32 * i:32 * i + 32,
                       288 * i:288 * i + 288].set(_f3(w2))
        b1bd = b1bd.at[32 * i:32 * i + 32].set(_col(b1))
        b2bd = b2bd.at[32 * i:32 * i + 32].set(_col(b2))

    wout = jnp.zeros((8, 288), BF16).at[0:1, :].set(_f3(conv_out_w))
    bout = jnp.zeros((8, 1), F32).at[0, 0].set(conv_out_b[0])

    operands = [xp, cmask, tmbd, pm, um,
                _f3(conv_in_w), _col(conv_in_b),
                *db1, *db2, *db3,
                gff_1x1_w.reshape(32, 96).astype(BF16), _col(gff_1x1_b),
                _f3(bc0_0_w), _col(bc0_0_b), _f3(bc0_1_w), _col(bc0_1_b),
                w1bd, b1bd, w2bd, b2bd, rbcat,
                _col(rb_b), wout, bout]

    grid = (N // NB,)

    def xmap(n):
        return (n, 0, 0)

    def wmap(n):
        return (0, 0)

    in_specs = [pl.BlockSpec((None, NB, SP), xmap)]
    in_specs += [pl.BlockSpec(op.shape, wmap) for op in operands[1:]]

    out = pl.pallas_call(
        _body,
        out_shape=jax.ShapeDtypeStruct((N // NB, NB, SP), F32),
        grid=grid,
        in_specs=in_specs,
        out_specs=pl.BlockSpec((None, NB, SP), xmap),
        scratch_shapes=(
            [pltpu.VMEM((96, BUFW), BF16)] * 2
            + [pltpu.VMEM((288, SP), BF16)] * 2
            + [pltpu.VMEM((64, SP), F32)] * 2
            + [pltpu.VMEM((96, SP), BF16)] * 2
            + [pltpu.VMEM((32, SP), F32)] * 2
            + [pltpu.VMEM((32, PTOT), F32)] * 2
            + [pltpu.VMEM((32, PTOT), BF16)] * 2
            + [pltpu.VMEM((1152, PTOT), BF16)] * 2
            + [pltpu.VMEM((128, TBW), BF16)] * 2
            + [pltpu.VMEM((128, TBW), BF16)] * 2
            + [pltpu.VMEM((128, PTOT), BF16)] * 2
        ),
        compiler_params=pltpu.CompilerParams(
            dimension_semantics=("parallel",)),
    )(*operands)

    out = out.reshape(N, H, WP)[:, :, 1:H + 1]
    return out.reshape(N, 1, H, H)
